# baseline (speedup 1.0000x reference)
import functools

import jax
import jax.numpy as jnp
from jax import lax
from jax.experimental import pallas as pl
from jax.experimental.pallas import tpu as pltpu
from jax.experimental.pallas import tpu_sc as plsc

N = 5000
P = 5120
C = 80
K = 100
KPAD = 104
BLK = 25
CONF = 0.1
IOU_T = 0.5
NWORK = 32


def _neg16():
    return jnp.full((16,), -jnp.inf, dtype=jnp.float32)


def _class_nms(ci, predT, kb_hbm, ks_hbm, kv_hbm,
               x1b, y1b, x2b, y2b, a2b, objb, sb, kbb, ksb, kvb,
               bufs_a, bufs_b):
    pltpu.sync_copy(predT.at[5 + ci], sb)

    csb, cx1b, cy1b, cx2b, cy2b, cab = bufs_a

    @plsc.parallel_loop(0, P, 16, unroll=2, carry=jnp.int32(0))
    def cnt(i, n):
        sl = pl.ds(i, 16)
        s = sb[sl] * objb[sl]
        msk = s > CONF
        dst = pl.ds(n, 16)
        plsc.store_compressed(csb.at[dst], s, mask=msk)
        plsc.store_compressed(cx1b.at[dst], x1b[sl], mask=msk)
        plsc.store_compressed(cy1b.at[dst], y1b[sl], mask=msk)
        plsc.store_compressed(cx2b.at[dst], x2b[sl], mask=msk)
        plsc.store_compressed(cy2b.at[dst], y2b[sl], mask=msk)
        plsc.store_compressed(cab.at[dst], a2b[sl], mask=msk)
        pc = plsc.all_reduce_population_count(msk)
        return n + pc[0]

    csb[pl.ds(cnt, 16)] = _neg16()
    cend = ((cnt + 15) // 16) * 16

    ksb[pl.ds(88, 16)] = jnp.zeros((16,), jnp.float32)
    kvb[pl.ds(88, 16)] = jnp.zeros((16,), jnp.int32)

    lane = lax.iota(jnp.int32, 16)
    lane0 = lane == 0

    def _hmax(x):
        for sh in (8, 4, 2, 1):
            x = jnp.maximum(x, x.at[lane ^ sh].get(mode="promise_in_bounds"))
        return x

    def _hmin_i32(x):
        for sh in (8, 4, 2, 1):
            x = jnp.minimum(x, x.at[lane ^ sh].get(mode="promise_in_bounds"))
        return x

    def _pick_block(k0, bufs, cend, carry0):
        csb, cx1b, cy1b, cx2b, cy2b, cab = bufs

        def pick(k, carry):
            bx1, by1, bx2, by2, ba = carry

            @plsc.parallel_loop(0, cend, 16, unroll=8,
                                carry=(_neg16(), jnp.zeros((16,), jnp.int32)))
            def scan_res(i, mc):
                m_v, b_v = mc
                sl = pl.ds(i, 16)
                s = csb[sl]
                px1 = cx1b[sl]
                py1 = cy1b[sl]
                px2 = cx2b[sl]
                py2 = cy2b[sl]
                pa = cab[sl]
                ix1 = jnp.maximum(bx1, px1)
                iy1 = jnp.maximum(by1, py1)
                ix2 = jnp.minimum(bx2, px2)
                iy2 = jnp.minimum(by2, py2)
                inter = jnp.maximum(ix2 - ix1, 0.0) * jnp.maximum(iy2 - iy1, 0.0)
                denom = ba + pa - inter + 1e-9
                s = jnp.where(inter > denom * IOU_T, _neg16(), s)
                csb[sl] = s
                upd = s > m_v
                m_v = jnp.where(upd, s, m_v)
                b_v = jnp.where(upd, jnp.full((16,), i, dtype=jnp.int32), b_v)
                return (m_v, b_v)

            m_v, b_v = scan_res

            ms = _hmax(m_v)
            gl = b_v + lane
            cand = jnp.where(m_v == ms, gl, jnp.full((16,), 2**30, jnp.int32))
            iv = _hmin_i32(cand)

            nx1 = plsc.load_gather(cx1b, [iv])
            ny1 = plsc.load_gather(cy1b, [iv])
            nx2 = plsc.load_gather(cx2b, [iv])
            ny2 = plsc.load_gather(cy2b, [iv])
            na = plsc.load_gather(cab, [iv])

            okv = ms != _neg16()

            rk = jnp.full((16,), k, dtype=jnp.int32)
            z16 = jnp.zeros((16,), jnp.float32)
            plsc.store_scatter(kbb, [rk, jnp.full((16,), 0, jnp.int32)],
                               jnp.where(okv, nx1, z16), mask=lane0)
            plsc.store_scatter(kbb, [rk, jnp.full((16,), 1, jnp.int32)],
                               jnp.where(okv, ny1, z16), mask=lane0)
            plsc.store_scatter(kbb, [rk, jnp.full((16,), 2, jnp.int32)],
                               jnp.where(okv, nx2, z16), mask=lane0)
            plsc.store_scatter(kbb, [rk, jnp.full((16,), 3, jnp.int32)],
                               jnp.where(okv, ny2, z16), mask=lane0)
            plsc.store_scatter(ksb, [rk], jnp.where(okv, ms, z16), mask=lane0)
            plsc.store_scatter(kvb, [rk],
                               jnp.where(okv, jnp.full((16,), 1, jnp.int32),
                                         jnp.zeros((16,), jnp.int32)),
                               mask=lane0)

            return (nx1, ny1, nx2, ny2, na)

        return lax.fori_loop(k0, k0 + BLK, pick, carry0)

    def _recompact(src, dst, cend_src):
        scs, sx1, sy1, sx2, sy2, sa = src
        dcs, dx1, dy1, dx2, dy2, da = dst

        @plsc.parallel_loop(0, cend_src, 16, unroll=2, carry=jnp.int32(0))
        def cnt2(i, n):
            sl = pl.ds(i, 16)
            s = scs[sl]
            msk = s != _neg16()
            dsl = pl.ds(n, 16)
            plsc.store_compressed(dcs.at[dsl], s, mask=msk)
            plsc.store_compressed(dx1.at[dsl], sx1[sl], mask=msk)
            plsc.store_compressed(dy1.at[dsl], sy1[sl], mask=msk)
            plsc.store_compressed(dx2.at[dsl], sx2[sl], mask=msk)
            plsc.store_compressed(dy2.at[dsl], sy2[sl], mask=msk)
            plsc.store_compressed(da.at[dsl], sa[sl], mask=msk)
            pc = plsc.all_reduce_population_count(msk)
            return n + pc[0]

        dcs[pl.ds(cnt2, 16)] = _neg16()
        return ((cnt2 + 15) // 16) * 16

    z = jnp.zeros((16,), jnp.float32)
    carry = (z, z, z, z, z)
    cur, other = bufs_a, bufs_b
    for blk in range(K // BLK):
        carry = _pick_block(blk * BLK, cur, cend, carry)
        if blk < K // BLK - 1:
            cend = _recompact(cur, other, cend)
            cur, other = other, cur

    pltpu.sync_copy(kbb, kb_hbm.at[ci])
    pltpu.sync_copy(ksb, ks_hbm.at[ci])
    pltpu.sync_copy(kvb, kv_hbm.at[ci])


def _make_sc_nms():
    mesh = plsc.VectorSubcoreMesh(core_axis_name="c", subcore_axis_name="s")

    @functools.partial(
        pl.kernel,
        mesh=mesh,
        compiler_params=pltpu.CompilerParams(needs_layout_passes=False),
        out_type=[
            jax.ShapeDtypeStruct((C, K, 4), jnp.float32),
            jax.ShapeDtypeStruct((C, KPAD), jnp.float32),
            jax.ShapeDtypeStruct((C, KPAD), jnp.int32),
        ],
        scratch_types=[
            pltpu.VMEM((P,), jnp.float32),
            pltpu.VMEM((P,), jnp.float32),
            pltpu.VMEM((P,), jnp.float32),
            pltpu.VMEM((P,), jnp.float32),
            pltpu.VMEM((P,), jnp.float32),
            pltpu.VMEM((P,), jnp.float32),
            pltpu.VMEM((P,), jnp.float32),
            pltpu.VMEM((K, 4), jnp.float32),
            pltpu.VMEM((KPAD,), jnp.float32),
            pltpu.VMEM((KPAD,), jnp.int32),
        ] + [pltpu.VMEM((P + 16,), jnp.float32)] * 12
          + [pltpu.SMEM((1,), jnp.int32)],
    )
    def sc_nms(predT, kb_hbm, ks_hbm, kv_hbm,
               x1b, y1b, x2b, y2b, a2b, objb, sb, kbb, ksb, kvb, *cbufs):
        sid = lax.axis_index("s")
        core = lax.axis_index("c")
        work = cbufs[12]

        @pl.when(sid == 0)
        def _():
            work[0] = 0

        pltpu.sync_copy(predT.at[0], x1b)
        pltpu.sync_copy(predT.at[1], y1b)
        pltpu.sync_copy(predT.at[2], x2b)
        pltpu.sync_copy(predT.at[3], y2b)
        pltpu.sync_copy(predT.at[4], objb)

        @plsc.parallel_loop(0, P, 16, unroll=4)
        def _box_g(i):
            sl = pl.ds(i, 16)
            cx = x1b[sl]
            cy = y1b[sl]
            w = x2b[sl]
            h = y2b[sl]
            xx1 = jnp.clip(cx - w / 2.0, 0.0, 1.0)
            yy1 = jnp.clip(cy - h / 2.0, 0.0, 1.0)
            xx2 = jnp.clip(cx + w / 2.0, 0.0, 1.0)
            yy2 = jnp.clip(cy + h / 2.0, 0.0, 1.0)
            area = jnp.maximum(xx2 - xx1, 0.0) * jnp.maximum(yy2 - yy1, 0.0)
            x1b[sl] = xx1
            y1b[sl] = yy1
            x2b[sl] = xx2
            y2b[sl] = yy2
            a2b[sl] = area

        args = (predT, kb_hbm, ks_hbm, kv_hbm,
                x1b, y1b, x2b, y2b, a2b, objb, sb, kbb, ksb, kvb,
                tuple(cbufs[:6]), tuple(cbufs[6:12]))

        plsc.subcore_barrier()
        ncls = C // 2

        def cond(j):
            return j < ncls

        def body(j):
            _class_nms(core * ncls + j, *args)
            return plsc.fetch_and_add(work.at[0], 1, subcore_id=0)

        lax.while_loop(cond, body,
                       plsc.fetch_and_add(work.at[0], 1, subcore_id=0))

    return sc_nms


_sc_nms = _make_sc_nms()


def kernel(pred, device):
    del device
    predT = jnp.transpose(pred)
    predT = jnp.pad(predT, ((0, 0), (0, P - N)))
    kb, ks, kv = _sc_nms(predT)
    labels = jnp.broadcast_to(jnp.arange(C, dtype=jnp.int32)[:, None], (C, K))
    return kb, labels, ks[:, :K], kv[:, :K].astype(bool)


# speedup vs baseline: 1.0734x; 1.0734x over previous
import functools

import jax
import jax.numpy as jnp
from jax import lax
from jax.experimental import pallas as pl
from jax.experimental.pallas import tpu as pltpu
from jax.experimental.pallas import tpu_sc as plsc

N = 5000
P = 5120
C = 80
K = 100
KPAD = 104
BLK = 20
CONF = 0.1
IOU_T = 0.5
NWORK = 32


def _neg16():
    return jnp.full((16,), -jnp.inf, dtype=jnp.float32)


def _class_nms(ci, predT, kb_hbm, ks_hbm, kv_hbm,
               x1b, y1b, x2b, y2b, a2b, objb, sb, kbb, ksb, kvb,
               bufs_a, bufs_b):
    pltpu.sync_copy(predT.at[5 + ci], sb)

    csb, cx1b, cy1b, cx2b, cy2b, cab = bufs_a

    @plsc.parallel_loop(0, P, 16, unroll=2, carry=jnp.int32(0))
    def cnt(i, n):
        sl = pl.ds(i, 16)
        s = sb[sl] * objb[sl]
        msk = s > CONF
        dst = pl.ds(n, 16)
        plsc.store_compressed(csb.at[dst], s, mask=msk)
        plsc.store_compressed(cx1b.at[dst], x1b[sl], mask=msk)
        plsc.store_compressed(cy1b.at[dst], y1b[sl], mask=msk)
        plsc.store_compressed(cx2b.at[dst], x2b[sl], mask=msk)
        plsc.store_compressed(cy2b.at[dst], y2b[sl], mask=msk)
        plsc.store_compressed(cab.at[dst], a2b[sl], mask=msk)
        pc = plsc.all_reduce_population_count(msk)
        return n + pc[0]

    csb[pl.ds(cnt, 16)] = _neg16()
    cend = ((cnt + 15) // 16) * 16

    ksb[pl.ds(88, 16)] = jnp.zeros((16,), jnp.float32)
    kvb[pl.ds(88, 16)] = jnp.zeros((16,), jnp.int32)

    lane = lax.iota(jnp.int32, 16)
    lane0 = lane == 0

    def _hmax(x):
        for sh in (8, 4, 2, 1):
            x = jnp.maximum(x, x.at[lane ^ sh].get(mode="promise_in_bounds"))
        return x

    def _hmin_i32(x):
        for sh in (8, 4, 2, 1):
            x = jnp.minimum(x, x.at[lane ^ sh].get(mode="promise_in_bounds"))
        return x

    def _pick_block(k0, bufs, cend, carry0):
        csb, cx1b, cy1b, cx2b, cy2b, cab = bufs

        def pick(k, carry):
            bx1, by1, bx2, by2, ba = carry

            @plsc.parallel_loop(0, cend, 16, unroll=4,
                                carry=(_neg16(), jnp.zeros((16,), jnp.int32)))
            def scan_res(i, mc):
                m_v, b_v = mc
                sl = pl.ds(i, 16)
                s = csb[sl]
                px1 = cx1b[sl]
                py1 = cy1b[sl]
                px2 = cx2b[sl]
                py2 = cy2b[sl]
                pa = cab[sl]
                ix1 = jnp.maximum(bx1, px1)
                iy1 = jnp.maximum(by1, py1)
                ix2 = jnp.minimum(bx2, px2)
                iy2 = jnp.minimum(by2, py2)
                inter = jnp.maximum(ix2 - ix1, 0.0) * jnp.maximum(iy2 - iy1, 0.0)
                denom = ba + pa - inter + 1e-9
                s = jnp.where(inter > denom * IOU_T, _neg16(), s)
                csb[sl] = s
                upd = s > m_v
                m_v = jnp.where(upd, s, m_v)
                b_v = jnp.where(upd, jnp.full((16,), i, dtype=jnp.int32), b_v)
                return (m_v, b_v)

            m_v, b_v = scan_res

            ms = _hmax(m_v)
            gl = b_v + lane
            cand = jnp.where(m_v == ms, gl, jnp.full((16,), 2**30, jnp.int32))
            iv = _hmin_i32(cand)

            nx1 = plsc.load_gather(cx1b, [iv])
            ny1 = plsc.load_gather(cy1b, [iv])
            nx2 = plsc.load_gather(cx2b, [iv])
            ny2 = plsc.load_gather(cy2b, [iv])
            na = plsc.load_gather(cab, [iv])

            okv = ms != _neg16()

            rk = jnp.full((16,), k, dtype=jnp.int32)
            z16 = jnp.zeros((16,), jnp.float32)
            plsc.store_scatter(kbb, [rk, jnp.full((16,), 0, jnp.int32)],
                               jnp.where(okv, nx1, z16), mask=lane0)
            plsc.store_scatter(kbb, [rk, jnp.full((16,), 1, jnp.int32)],
                               jnp.where(okv, ny1, z16), mask=lane0)
            plsc.store_scatter(kbb, [rk, jnp.full((16,), 2, jnp.int32)],
                               jnp.where(okv, nx2, z16), mask=lane0)
            plsc.store_scatter(kbb, [rk, jnp.full((16,), 3, jnp.int32)],
                               jnp.where(okv, ny2, z16), mask=lane0)
            plsc.store_scatter(ksb, [rk], jnp.where(okv, ms, z16), mask=lane0)
            plsc.store_scatter(kvb, [rk],
                               jnp.where(okv, jnp.full((16,), 1, jnp.int32),
                                         jnp.zeros((16,), jnp.int32)),
                               mask=lane0)

            return (nx1, ny1, nx2, ny2, na)

        return lax.fori_loop(k0, k0 + BLK, pick, carry0)

    def _recompact(src, dst, cend_src):
        scs, sx1, sy1, sx2, sy2, sa = src
        dcs, dx1, dy1, dx2, dy2, da = dst

        @plsc.parallel_loop(0, cend_src, 16, unroll=2, carry=jnp.int32(0))
        def cnt2(i, n):
            sl = pl.ds(i, 16)
            s = scs[sl]
            msk = s != _neg16()
            dsl = pl.ds(n, 16)
            plsc.store_compressed(dcs.at[dsl], s, mask=msk)
            plsc.store_compressed(dx1.at[dsl], sx1[sl], mask=msk)
            plsc.store_compressed(dy1.at[dsl], sy1[sl], mask=msk)
            plsc.store_compressed(dx2.at[dsl], sx2[sl], mask=msk)
            plsc.store_compressed(dy2.at[dsl], sy2[sl], mask=msk)
            plsc.store_compressed(da.at[dsl], sa[sl], mask=msk)
            pc = plsc.all_reduce_population_count(msk)
            return n + pc[0]

        dcs[pl.ds(cnt2, 16)] = _neg16()
        return ((cnt2 + 15) // 16) * 16

    z = jnp.zeros((16,), jnp.float32)
    carry = (z, z, z, z, z)
    cur, other = bufs_a, bufs_b
    for blk in range(K // BLK):
        carry = _pick_block(blk * BLK, cur, cend, carry)
        if blk < K // BLK - 1:
            cend = _recompact(cur, other, cend)
            cur, other = other, cur

    pltpu.sync_copy(kbb, kb_hbm.at[ci])
    pltpu.sync_copy(ksb, ks_hbm.at[ci])
    pltpu.sync_copy(kvb, kv_hbm.at[ci])


def _make_sc_nms():
    mesh = plsc.VectorSubcoreMesh(core_axis_name="c", subcore_axis_name="s")

    @functools.partial(
        pl.kernel,
        mesh=mesh,
        compiler_params=pltpu.CompilerParams(needs_layout_passes=False),
        out_type=[
            jax.ShapeDtypeStruct((C, K, 4), jnp.float32),
            jax.ShapeDtypeStruct((C, KPAD), jnp.float32),
            jax.ShapeDtypeStruct((C, KPAD), jnp.int32),
        ],
        scratch_types=[
            pltpu.VMEM((P,), jnp.float32),
            pltpu.VMEM((P,), jnp.float32),
            pltpu.VMEM((P,), jnp.float32),
            pltpu.VMEM((P,), jnp.float32),
            pltpu.VMEM((P,), jnp.float32),
            pltpu.VMEM((P,), jnp.float32),
            pltpu.VMEM((P,), jnp.float32),
            pltpu.VMEM((K, 4), jnp.float32),
            pltpu.VMEM((KPAD,), jnp.float32),
            pltpu.VMEM((KPAD,), jnp.int32),
        ] + [pltpu.VMEM((P + 16,), jnp.float32)] * 12
          + [pltpu.SMEM((1,), jnp.int32)],
    )
    def sc_nms(predT, kb_hbm, ks_hbm, kv_hbm,
               x1b, y1b, x2b, y2b, a2b, objb, sb, kbb, ksb, kvb, *cbufs):
        sid = lax.axis_index("s")
        core = lax.axis_index("c")
        work = cbufs[12]

        @pl.when(sid == 0)
        def _():
            work[0] = 0

        pltpu.sync_copy(predT.at[0], x1b)
        pltpu.sync_copy(predT.at[1], y1b)
        pltpu.sync_copy(predT.at[2], x2b)
        pltpu.sync_copy(predT.at[3], y2b)
        pltpu.sync_copy(predT.at[4], objb)

        @plsc.parallel_loop(0, P, 16, unroll=4)
        def _box_g(i):
            sl = pl.ds(i, 16)
            cx = x1b[sl]
            cy = y1b[sl]
            w = x2b[sl]
            h = y2b[sl]
            xx1 = jnp.clip(cx - w / 2.0, 0.0, 1.0)
            yy1 = jnp.clip(cy - h / 2.0, 0.0, 1.0)
            xx2 = jnp.clip(cx + w / 2.0, 0.0, 1.0)
            yy2 = jnp.clip(cy + h / 2.0, 0.0, 1.0)
            area = jnp.maximum(xx2 - xx1, 0.0) * jnp.maximum(yy2 - yy1, 0.0)
            x1b[sl] = xx1
            y1b[sl] = yy1
            x2b[sl] = xx2
            y2b[sl] = yy2
            a2b[sl] = area

        args = (predT, kb_hbm, ks_hbm, kv_hbm,
                x1b, y1b, x2b, y2b, a2b, objb, sb, kbb, ksb, kvb,
                tuple(cbufs[:6]), tuple(cbufs[6:12]))

        plsc.subcore_barrier()
        ncls = C // 2

        def cond(j):
            return j < ncls

        def body(j):
            _class_nms(core * ncls + j, *args)
            return plsc.fetch_and_add(work.at[0], 1, subcore_id=0)

        lax.while_loop(cond, body,
                       plsc.fetch_and_add(work.at[0], 1, subcore_id=0))

    return sc_nms


_sc_nms = _make_sc_nms()


def kernel(pred, device):
    del device
    predT = jnp.transpose(pred)
    predT = jnp.pad(predT, ((0, 0), (0, P - N)))
    kb, ks, kv = _sc_nms(predT)
    labels = jnp.broadcast_to(jnp.arange(C, dtype=jnp.int32)[:, None], (C, K))
    return kb, labels, ks[:, :K], kv[:, :K].astype(bool)


# speedup vs baseline: 2.2339x; 2.0812x over previous
import functools

import jax
import jax.numpy as jnp
from jax import lax
from jax.experimental import pallas as pl
from jax.experimental.pallas import tpu as pltpu
from jax.experimental.pallas import tpu_sc as plsc

N = 5000
P = 5120
C = 80
K = 100
BLK = 20
CONF = 0.1
IOU_T = 0.5
PIV = 0.55


def _neg16():
    return jnp.full((16,), -jnp.inf, dtype=jnp.float32)


def _run_nms(thresh, predT, kb_hbm,
             x1b, y1b, x2b, y2b, a2b, objb, sb, kbb,
             bufs_a, bufs_b):
    csb, cx1b, cy1b, cx2b, cy2b, cab = bufs_a

    @plsc.parallel_loop(0, P, 16, unroll=2, carry=jnp.int32(0))
    def cnt(i, n):
        sl = pl.ds(i, 16)
        s = sb[sl] * objb[sl]
        msk = s > thresh
        dst = pl.ds(n, 16)
        plsc.store_compressed(csb.at[dst], s, mask=msk)
        plsc.store_compressed(cx1b.at[dst], x1b[sl], mask=msk)
        plsc.store_compressed(cy1b.at[dst], y1b[sl], mask=msk)
        plsc.store_compressed(cx2b.at[dst], x2b[sl], mask=msk)
        plsc.store_compressed(cy2b.at[dst], y2b[sl], mask=msk)
        plsc.store_compressed(cab.at[dst], a2b[sl], mask=msk)
        pc = plsc.all_reduce_population_count(msk)
        return n + pc[0]

    csb[pl.ds(cnt, 16)] = _neg16()
    cend = ((cnt + 15) // 16) * 16

    lane = lax.iota(jnp.int32, 16)
    lane6 = lane < 6
    le0 = lane == 0
    le1 = lane == 1
    le2 = lane == 2
    le3 = lane == 3
    le4 = lane == 4

    def _hmax(x):
        for sh in (8, 4, 2, 1):
            x = jnp.maximum(x, x.at[lane ^ sh].get(mode="promise_in_bounds"))
        return x

    def _hmin_i32(x):
        for sh in (8, 4, 2, 1):
            x = jnp.minimum(x, x.at[lane ^ sh].get(mode="promise_in_bounds"))
        return x

    def _pick_block(k0, bufs, cend, carry0):
        csb, cx1b, cy1b, cx2b, cy2b, cab = bufs

        def pick(k, carry):
            bx1, by1, bx2, by2, ba, mn = carry

            @plsc.parallel_loop(0, cend, 16, unroll=4,
                                carry=(_neg16(), jnp.zeros((16,), jnp.int32)))
            def scan_res(i, mc):
                m_v, b_v = mc
                sl = pl.ds(i, 16)
                s = csb[sl]
                px1 = cx1b[sl]
                py1 = cy1b[sl]
                px2 = cx2b[sl]
                py2 = cy2b[sl]
                pa = cab[sl]
                ix1 = jnp.maximum(bx1, px1)
                iy1 = jnp.maximum(by1, py1)
                ix2 = jnp.minimum(bx2, px2)
                iy2 = jnp.minimum(by2, py2)
                inter = (jnp.maximum(ix2 - ix1, 0.0)
                         * jnp.maximum(iy2 - iy1, 0.0))
                denom = ba + pa - inter + 1e-9
                s = jnp.where(inter > denom * IOU_T, _neg16(), s)
                csb[sl] = s
                upd = s > m_v
                m_v = jnp.where(upd, s, m_v)
                b_v = jnp.where(upd, jnp.full((16,), i, dtype=jnp.int32), b_v)
                return (m_v, b_v)

            m_v, b_v = scan_res

            ms = _hmax(m_v)
            gl = b_v + lane
            cand = jnp.where(m_v == ms, gl, jnp.full((16,), 2**30, jnp.int32))
            iv = _hmin_i32(cand)

            nx1 = plsc.load_gather(cx1b, [iv])
            ny1 = plsc.load_gather(cy1b, [iv])
            nx2 = plsc.load_gather(cx2b, [iv])
            ny2 = plsc.load_gather(cy2b, [iv])
            na = plsc.load_gather(cab, [iv])

            okv = ms != _neg16()

            val = jnp.where(le0, nx1,
                            jnp.where(le1, ny1,
                                      jnp.where(le2, nx2,
                                                jnp.where(le3, ny2,
                                                          jnp.where(le4, ms,
                                                                    1.0)))))
            val = jnp.where(okv, val, jnp.zeros((16,), jnp.float32))
            plsc.store_scatter(kbb, [jnp.full((16,), k, dtype=jnp.int32),
                                     lane], val, mask=lane6)

            return (nx1, ny1, nx2, ny2, na, jnp.minimum(mn, ms))

        return lax.fori_loop(k0, k0 + BLK, pick, carry0)

    def _recompact(src, dst, cend_src):
        scs, sx1, sy1, sx2, sy2, sa = src
        dcs, dx1, dy1, dx2, dy2, da = dst

        @plsc.parallel_loop(0, cend_src, 16, unroll=2, carry=jnp.int32(0))
        def cnt2(i, n):
            sl = pl.ds(i, 16)
            s = scs[sl]
            msk = s != _neg16()
            dsl = pl.ds(n, 16)
            plsc.store_compressed(dcs.at[dsl], s, mask=msk)
            plsc.store_compressed(dx1.at[dsl], sx1[sl], mask=msk)
            plsc.store_compressed(dy1.at[dsl], sy1[sl], mask=msk)
            plsc.store_compressed(dx2.at[dsl], sx2[sl], mask=msk)
            plsc.store_compressed(dy2.at[dsl], sy2[sl], mask=msk)
            plsc.store_compressed(da.at[dsl], sa[sl], mask=msk)
            pc = plsc.all_reduce_population_count(msk)
            return n + pc[0]

        dcs[pl.ds(cnt2, 16)] = _neg16()
        return ((cnt2 + 15) // 16) * 16

    z = jnp.zeros((16,), jnp.float32)
    carry = (z, z, z, z, z, jnp.full((16,), jnp.inf, dtype=jnp.float32))
    cur, other = bufs_a, bufs_b
    for blk in range(K // BLK):
        carry = _pick_block(blk * BLK, cur, cend, carry)
        if blk < K // BLK - 1:
            cend = _recompact(cur, other, cend)
            cur, other = other, cur
    return carry[5]


def _class_nms(ci, predT, kb_hbm,
               x1b, y1b, x2b, y2b, a2b, objb, sb, kbb, fbuf,
               bufs_a, bufs_b):
    pltpu.sync_copy(predT.at[5 + ci], sb)

    args = (predT, kb_hbm, x1b, y1b, x2b, y2b, a2b, objb, sb, kbb,
            bufs_a, bufs_b)

    mn = _run_nms(PIV, *args)

    @pl.when(mn[0] <= PIV)
    def _():
        _run_nms(CONF, *args)

    pltpu.sync_copy(kbb, kb_hbm.at[ci])


def _make_sc_nms():
    mesh = plsc.VectorSubcoreMesh(core_axis_name="c", subcore_axis_name="s")

    @functools.partial(
        pl.kernel,
        mesh=mesh,
        compiler_params=pltpu.CompilerParams(needs_layout_passes=False),
        out_type=jax.ShapeDtypeStruct((C, K, 8), jnp.float32),
        scratch_types=[
            pltpu.VMEM((P,), jnp.float32),
            pltpu.VMEM((P,), jnp.float32),
            pltpu.VMEM((P,), jnp.float32),
            pltpu.VMEM((P,), jnp.float32),
            pltpu.VMEM((P,), jnp.float32),
            pltpu.VMEM((P,), jnp.float32),
            pltpu.VMEM((P,), jnp.float32),
            pltpu.VMEM((K, 8), jnp.float32),
            pltpu.VMEM((16,), jnp.float32),
        ] + [pltpu.VMEM((P + 16,), jnp.float32)] * 12
          + [pltpu.SMEM((1,), jnp.int32)],
    )
    def sc_nms(predT, kb_hbm,
               x1b, y1b, x2b, y2b, a2b, objb, sb, kbb, fbuf, *cbufs):
        sid = lax.axis_index("s")
        core = lax.axis_index("c")
        work = cbufs[12]

        @pl.when(sid == 0)
        def _():
            work[0] = 0

        pltpu.sync_copy(predT.at[0], x1b)
        pltpu.sync_copy(predT.at[1], y1b)
        pltpu.sync_copy(predT.at[2], x2b)
        pltpu.sync_copy(predT.at[3], y2b)
        pltpu.sync_copy(predT.at[4], objb)

        @plsc.parallel_loop(0, P, 16, unroll=4)
        def _box_g(i):
            sl = pl.ds(i, 16)
            cx = x1b[sl]
            cy = y1b[sl]
            w = x2b[sl]
            h = y2b[sl]
            xx1 = jnp.clip(cx - w / 2.0, 0.0, 1.0)
            yy1 = jnp.clip(cy - h / 2.0, 0.0, 1.0)
            xx2 = jnp.clip(cx + w / 2.0, 0.0, 1.0)
            yy2 = jnp.clip(cy + h / 2.0, 0.0, 1.0)
            area = jnp.maximum(xx2 - xx1, 0.0) * jnp.maximum(yy2 - yy1, 0.0)
            x1b[sl] = xx1
            y1b[sl] = yy1
            x2b[sl] = xx2
            y2b[sl] = yy2
            a2b[sl] = area

        args = (predT, kb_hbm, x1b, y1b, x2b, y2b, a2b, objb, sb, kbb, fbuf,
                tuple(cbufs[:6]), tuple(cbufs[6:12]))

        plsc.subcore_barrier()
        ncls = C // 2

        def cond(j):
            return j < ncls

        def body(j):
            _class_nms(core * ncls + j, *args)
            return plsc.fetch_and_add(work.at[0], 1, subcore_id=0)

        lax.while_loop(cond, body,
                       plsc.fetch_and_add(work.at[0], 1, subcore_id=0))

    return sc_nms


_sc_nms = _make_sc_nms()


def kernel(pred, device):
    del device
    predT = jnp.transpose(pred)
    predT = jnp.pad(predT, ((0, 0), (0, P - N)))
    kb8 = _sc_nms(predT)
    labels = jnp.broadcast_to(jnp.arange(C, dtype=jnp.int32)[:, None], (C, K))
    return (kb8[:, :, :4], labels, kb8[:, :, 4], kb8[:, :, 5].astype(bool))


# speedup vs baseline: 2.2821x; 1.0216x over previous
import functools

import jax
import jax.numpy as jnp
from jax import lax
from jax.experimental import pallas as pl
from jax.experimental.pallas import tpu as pltpu
from jax.experimental.pallas import tpu_sc as plsc

N = 5000
P = 5120
C = 80
K = 100
BLK = 50
CONF = 0.1
IOU_T = 0.5
PIV = 0.55


def _neg16():
    return jnp.full((16,), -jnp.inf, dtype=jnp.float32)


def _run_nms(thresh, predT, kb_hbm,
             x1b, y1b, x2b, y2b, a2b, objb, sb, kbb,
             bufs_a, bufs_b):
    csb, cx1b, cy1b, cx2b, cy2b, cab = bufs_a

    @plsc.parallel_loop(0, P, 16, unroll=2, carry=jnp.int32(0))
    def cnt(i, n):
        sl = pl.ds(i, 16)
        s = sb[sl] * objb[sl]
        msk = s > thresh
        dst = pl.ds(n, 16)
        plsc.store_compressed(csb.at[dst], s, mask=msk)
        plsc.store_compressed(cx1b.at[dst], x1b[sl], mask=msk)
        plsc.store_compressed(cy1b.at[dst], y1b[sl], mask=msk)
        plsc.store_compressed(cx2b.at[dst], x2b[sl], mask=msk)
        plsc.store_compressed(cy2b.at[dst], y2b[sl], mask=msk)
        plsc.store_compressed(cab.at[dst], a2b[sl], mask=msk)
        pc = plsc.all_reduce_population_count(msk)
        return n + pc[0]

    csb[pl.ds(cnt, 16)] = _neg16()
    cend = ((cnt + 15) // 16) * 16

    lane = lax.iota(jnp.int32, 16)
    lane6 = lane < 6
    le0 = lane == 0
    le1 = lane == 1
    le2 = lane == 2
    le3 = lane == 3
    le4 = lane == 4

    def _hmax(x):
        for sh in (8, 4, 2, 1):
            x = jnp.maximum(x, x.at[lane ^ sh].get(mode="promise_in_bounds"))
        return x

    def _hmin_i32(x):
        for sh in (8, 4, 2, 1):
            x = jnp.minimum(x, x.at[lane ^ sh].get(mode="promise_in_bounds"))
        return x

    def _pick_block(k0, bufs, cend, carry0):
        csb, cx1b, cy1b, cx2b, cy2b, cab = bufs

        def pick(k, carry):
            bx1, by1, bx2, by2, ba, mn = carry

            @plsc.parallel_loop(0, cend, 16, unroll=4,
                                carry=(_neg16(), jnp.zeros((16,), jnp.int32)))
            def scan_res(i, mc):
                m_v, b_v = mc
                sl = pl.ds(i, 16)
                s = csb[sl]
                px1 = cx1b[sl]
                py1 = cy1b[sl]
                px2 = cx2b[sl]
                py2 = cy2b[sl]
                pa = cab[sl]
                ix1 = jnp.maximum(bx1, px1)
                iy1 = jnp.maximum(by1, py1)
                ix2 = jnp.minimum(bx2, px2)
                iy2 = jnp.minimum(by2, py2)
                inter = (jnp.maximum(ix2 - ix1, 0.0)
                         * jnp.maximum(iy2 - iy1, 0.0))
                denom = ba + pa - inter + 1e-9
                s = jnp.where(inter > denom * IOU_T, _neg16(), s)
                csb[sl] = s
                upd = s > m_v
                m_v = jnp.where(upd, s, m_v)
                b_v = jnp.where(upd, jnp.full((16,), i, dtype=jnp.int32), b_v)
                return (m_v, b_v)

            m_v, b_v = scan_res

            ms = _hmax(m_v)
            gl = b_v + lane
            cand = jnp.where(m_v == ms, gl, jnp.full((16,), 2**30, jnp.int32))
            iv = _hmin_i32(cand)

            nx1 = plsc.load_gather(cx1b, [iv])
            ny1 = plsc.load_gather(cy1b, [iv])
            nx2 = plsc.load_gather(cx2b, [iv])
            ny2 = plsc.load_gather(cy2b, [iv])
            na = plsc.load_gather(cab, [iv])

            okv = ms != _neg16()

            val = jnp.where(le0, nx1,
                            jnp.where(le1, ny1,
                                      jnp.where(le2, nx2,
                                                jnp.where(le3, ny2,
                                                          jnp.where(le4, ms,
                                                                    1.0)))))
            val = jnp.where(okv, val, jnp.zeros((16,), jnp.float32))
            plsc.store_scatter(kbb, [jnp.full((16,), k, dtype=jnp.int32),
                                     lane], val, mask=lane6)

            return (nx1, ny1, nx2, ny2, na, jnp.minimum(mn, ms))

        return lax.fori_loop(k0, k0 + BLK, pick, carry0)

    def _recompact(src, dst, cend_src):
        scs, sx1, sy1, sx2, sy2, sa = src
        dcs, dx1, dy1, dx2, dy2, da = dst

        @plsc.parallel_loop(0, cend_src, 16, unroll=2, carry=jnp.int32(0))
        def cnt2(i, n):
            sl = pl.ds(i, 16)
            s = scs[sl]
            msk = s != _neg16()
            dsl = pl.ds(n, 16)
            plsc.store_compressed(dcs.at[dsl], s, mask=msk)
            plsc.store_compressed(dx1.at[dsl], sx1[sl], mask=msk)
            plsc.store_compressed(dy1.at[dsl], sy1[sl], mask=msk)
            plsc.store_compressed(dx2.at[dsl], sx2[sl], mask=msk)
            plsc.store_compressed(dy2.at[dsl], sy2[sl], mask=msk)
            plsc.store_compressed(da.at[dsl], sa[sl], mask=msk)
            pc = plsc.all_reduce_population_count(msk)
            return n + pc[0]

        dcs[pl.ds(cnt2, 16)] = _neg16()
        return ((cnt2 + 15) // 16) * 16

    z = jnp.zeros((16,), jnp.float32)
    carry = (z, z, z, z, z, jnp.full((16,), jnp.inf, dtype=jnp.float32))
    cur, other = bufs_a, bufs_b
    for blk in range(K // BLK):
        carry = _pick_block(blk * BLK, cur, cend, carry)
        if blk < K // BLK - 1:
            cend = _recompact(cur, other, cend)
            cur, other = other, cur
    return carry[5]


def _class_nms(ci, predT, kb_hbm,
               x1b, y1b, x2b, y2b, a2b, objb, sb, kbb, fbuf,
               bufs_a, bufs_b):
    pltpu.sync_copy(predT.at[5 + ci], sb)

    args = (predT, kb_hbm, x1b, y1b, x2b, y2b, a2b, objb, sb, kbb,
            bufs_a, bufs_b)

    mn = _run_nms(PIV, *args)

    @pl.when(mn[0] <= PIV)
    def _():
        _run_nms(CONF, *args)

    pltpu.sync_copy(kbb, kb_hbm.at[ci])


def _make_sc_nms():
    mesh = plsc.VectorSubcoreMesh(core_axis_name="c", subcore_axis_name="s")

    @functools.partial(
        pl.kernel,
        mesh=mesh,
        compiler_params=pltpu.CompilerParams(needs_layout_passes=False),
        out_type=jax.ShapeDtypeStruct((C, K, 8), jnp.float32),
        scratch_types=[
            pltpu.VMEM((P,), jnp.float32),
            pltpu.VMEM((P,), jnp.float32),
            pltpu.VMEM((P,), jnp.float32),
            pltpu.VMEM((P,), jnp.float32),
            pltpu.VMEM((P,), jnp.float32),
            pltpu.VMEM((P,), jnp.float32),
            pltpu.VMEM((P,), jnp.float32),
            pltpu.VMEM((K, 8), jnp.float32),
            pltpu.VMEM((16,), jnp.float32),
        ] + [pltpu.VMEM((P + 16,), jnp.float32)] * 12
          + [pltpu.SMEM((1,), jnp.int32)],
    )
    def sc_nms(predT, kb_hbm,
               x1b, y1b, x2b, y2b, a2b, objb, sb, kbb, fbuf, *cbufs):
        sid = lax.axis_index("s")
        core = lax.axis_index("c")
        work = cbufs[12]

        @pl.when(sid == 0)
        def _():
            work[0] = 0

        pltpu.sync_copy(predT.at[0], x1b)
        pltpu.sync_copy(predT.at[1], y1b)
        pltpu.sync_copy(predT.at[2], x2b)
        pltpu.sync_copy(predT.at[3], y2b)
        pltpu.sync_copy(predT.at[4], objb)

        @plsc.parallel_loop(0, P, 16, unroll=4)
        def _box_g(i):
            sl = pl.ds(i, 16)
            cx = x1b[sl]
            cy = y1b[sl]
            w = x2b[sl]
            h = y2b[sl]
            xx1 = jnp.clip(cx - w / 2.0, 0.0, 1.0)
            yy1 = jnp.clip(cy - h / 2.0, 0.0, 1.0)
            xx2 = jnp.clip(cx + w / 2.0, 0.0, 1.0)
            yy2 = jnp.clip(cy + h / 2.0, 0.0, 1.0)
            area = jnp.maximum(xx2 - xx1, 0.0) * jnp.maximum(yy2 - yy1, 0.0)
            x1b[sl] = xx1
            y1b[sl] = yy1
            x2b[sl] = xx2
            y2b[sl] = yy2
            a2b[sl] = area

        args = (predT, kb_hbm, x1b, y1b, x2b, y2b, a2b, objb, sb, kbb, fbuf,
                tuple(cbufs[:6]), tuple(cbufs[6:12]))

        plsc.subcore_barrier()
        ncls = C // 2

        def cond(j):
            return j < ncls

        def body(j):
            _class_nms(core * ncls + j, *args)
            return plsc.fetch_and_add(work.at[0], 1, subcore_id=0)

        lax.while_loop(cond, body,
                       plsc.fetch_and_add(work.at[0], 1, subcore_id=0))

    return sc_nms


_sc_nms = _make_sc_nms()


def kernel(pred, device):
    del device
    predT = jnp.transpose(pred)
    predT = jnp.pad(predT, ((0, 0), (0, P - N)))
    kb8 = _sc_nms(predT)
    labels = jnp.broadcast_to(jnp.arange(C, dtype=jnp.int32)[:, None], (C, K))
    return (kb8[:, :, :4], labels, kb8[:, :, 4], kb8[:, :, 5].astype(bool))


# speedup vs baseline: 2.5689x; 1.1257x over previous
import functools

import jax
import jax.numpy as jnp
from jax import lax
from jax.experimental import pallas as pl
from jax.experimental.pallas import tpu as pltpu
from jax.experimental.pallas import tpu_sc as plsc

N = 5000
P = 5120
C = 80
K = 100
BLK = 50
CONF = 0.1
IOU_T = 0.5
PIV = 0.65


def _neg16():
    return jnp.full((16,), -jnp.inf, dtype=jnp.float32)


def _run_nms(thresh, predT, kb_hbm,
             x1b, y1b, x2b, y2b, a2b, objb, sb, kbb,
             bufs_a, bufs_b):
    csb, cx1b, cy1b, cx2b, cy2b, cab = bufs_a

    @plsc.parallel_loop(0, P, 16, unroll=2, carry=jnp.int32(0))
    def cnt(i, n):
        sl = pl.ds(i, 16)
        s = sb[sl] * objb[sl]
        msk = s > thresh
        dst = pl.ds(n, 16)
        plsc.store_compressed(csb.at[dst], s, mask=msk)
        plsc.store_compressed(cx1b.at[dst], x1b[sl], mask=msk)
        plsc.store_compressed(cy1b.at[dst], y1b[sl], mask=msk)
        plsc.store_compressed(cx2b.at[dst], x2b[sl], mask=msk)
        plsc.store_compressed(cy2b.at[dst], y2b[sl], mask=msk)
        plsc.store_compressed(cab.at[dst], a2b[sl], mask=msk)
        pc = plsc.all_reduce_population_count(msk)
        return n + pc[0]

    csb[pl.ds(cnt, 16)] = _neg16()
    cend = ((cnt + 15) // 16) * 16

    lane = lax.iota(jnp.int32, 16)
    lane6 = lane < 6
    le0 = lane == 0
    le1 = lane == 1
    le2 = lane == 2
    le3 = lane == 3
    le4 = lane == 4

    def _hmax(x):
        for sh in (8, 4, 2, 1):
            x = jnp.maximum(x, x.at[lane ^ sh].get(mode="promise_in_bounds"))
        return x

    def _hmin_i32(x):
        for sh in (8, 4, 2, 1):
            x = jnp.minimum(x, x.at[lane ^ sh].get(mode="promise_in_bounds"))
        return x

    def _pick_block(k0, bufs, cend, carry0):
        csb, cx1b, cy1b, cx2b, cy2b, cab = bufs

        def pick(k, carry):
            bx1, by1, bx2, by2, ba, mn = carry

            @plsc.parallel_loop(0, cend, 16, unroll=4,
                                carry=(_neg16(), jnp.zeros((16,), jnp.int32)))
            def scan_res(i, mc):
                m_v, b_v = mc
                sl = pl.ds(i, 16)
                s = csb[sl]
                px1 = cx1b[sl]
                py1 = cy1b[sl]
                px2 = cx2b[sl]
                py2 = cy2b[sl]
                pa = cab[sl]
                ix1 = jnp.maximum(bx1, px1)
                iy1 = jnp.maximum(by1, py1)
                ix2 = jnp.minimum(bx2, px2)
                iy2 = jnp.minimum(by2, py2)
                inter = (jnp.maximum(ix2 - ix1, 0.0)
                         * jnp.maximum(iy2 - iy1, 0.0))
                denom = ba + pa - inter + 1e-9
                s = jnp.where(inter > denom * IOU_T, _neg16(), s)
                csb[sl] = s
                upd = s > m_v
                m_v = jnp.where(upd, s, m_v)
                b_v = jnp.where(upd, jnp.full((16,), i, dtype=jnp.int32), b_v)
                return (m_v, b_v)

            m_v, b_v = scan_res

            ms = _hmax(m_v)
            gl = b_v + lane
            cand = jnp.where(m_v == ms, gl, jnp.full((16,), 2**30, jnp.int32))
            iv = _hmin_i32(cand)

            nx1 = plsc.load_gather(cx1b, [iv])
            ny1 = plsc.load_gather(cy1b, [iv])
            nx2 = plsc.load_gather(cx2b, [iv])
            ny2 = plsc.load_gather(cy2b, [iv])
            na = plsc.load_gather(cab, [iv])

            okv = ms != _neg16()

            val = jnp.where(le0, nx1,
                            jnp.where(le1, ny1,
                                      jnp.where(le2, nx2,
                                                jnp.where(le3, ny2,
                                                          jnp.where(le4, ms,
                                                                    1.0)))))
            val = jnp.where(okv, val, jnp.zeros((16,), jnp.float32))
            plsc.store_scatter(kbb, [jnp.full((16,), k, dtype=jnp.int32),
                                     lane], val, mask=lane6)

            return (nx1, ny1, nx2, ny2, na, jnp.minimum(mn, ms))

        return lax.fori_loop(k0, k0 + BLK, pick, carry0)

    def _recompact(src, dst, cend_src):
        scs, sx1, sy1, sx2, sy2, sa = src
        dcs, dx1, dy1, dx2, dy2, da = dst

        @plsc.parallel_loop(0, cend_src, 16, unroll=2, carry=jnp.int32(0))
        def cnt2(i, n):
            sl = pl.ds(i, 16)
            s = scs[sl]
            msk = s != _neg16()
            dsl = pl.ds(n, 16)
            plsc.store_compressed(dcs.at[dsl], s, mask=msk)
            plsc.store_compressed(dx1.at[dsl], sx1[sl], mask=msk)
            plsc.store_compressed(dy1.at[dsl], sy1[sl], mask=msk)
            plsc.store_compressed(dx2.at[dsl], sx2[sl], mask=msk)
            plsc.store_compressed(dy2.at[dsl], sy2[sl], mask=msk)
            plsc.store_compressed(da.at[dsl], sa[sl], mask=msk)
            pc = plsc.all_reduce_population_count(msk)
            return n + pc[0]

        dcs[pl.ds(cnt2, 16)] = _neg16()
        return ((cnt2 + 15) // 16) * 16

    z = jnp.zeros((16,), jnp.float32)
    carry = (z, z, z, z, z, jnp.full((16,), jnp.inf, dtype=jnp.float32))
    cur, other = bufs_a, bufs_b
    for blk in range(K // BLK):
        carry = _pick_block(blk * BLK, cur, cend, carry)
        if blk < K // BLK - 1:
            cend = _recompact(cur, other, cend)
            cur, other = other, cur
    return carry[5]


def _class_nms(ci, predT, kb_hbm,
               x1b, y1b, x2b, y2b, a2b, objb, sb, kbb, fbuf,
               bufs_a, bufs_b):
    pltpu.sync_copy(predT.at[5 + ci], sb)

    args = (predT, kb_hbm, x1b, y1b, x2b, y2b, a2b, objb, sb, kbb,
            bufs_a, bufs_b)

    mn = _run_nms(PIV, *args)

    @pl.when(mn[0] <= PIV)
    def _():
        _run_nms(CONF, *args)

    pltpu.sync_copy(kbb, kb_hbm.at[ci])


def _make_sc_nms():
    mesh = plsc.VectorSubcoreMesh(core_axis_name="c", subcore_axis_name="s")

    @functools.partial(
        pl.kernel,
        mesh=mesh,
        compiler_params=pltpu.CompilerParams(needs_layout_passes=False),
        out_type=jax.ShapeDtypeStruct((C, K, 8), jnp.float32),
        scratch_types=[
            pltpu.VMEM((P,), jnp.float32),
            pltpu.VMEM((P,), jnp.float32),
            pltpu.VMEM((P,), jnp.float32),
            pltpu.VMEM((P,), jnp.float32),
            pltpu.VMEM((P,), jnp.float32),
            pltpu.VMEM((P,), jnp.float32),
            pltpu.VMEM((P,), jnp.float32),
            pltpu.VMEM((K, 8), jnp.float32),
            pltpu.VMEM((16,), jnp.float32),
        ] + [pltpu.VMEM((P + 16,), jnp.float32)] * 12
          + [pltpu.SMEM((1,), jnp.int32)],
    )
    def sc_nms(predT, kb_hbm,
               x1b, y1b, x2b, y2b, a2b, objb, sb, kbb, fbuf, *cbufs):
        sid = lax.axis_index("s")
        core = lax.axis_index("c")
        work = cbufs[12]

        @pl.when(sid == 0)
        def _():
            work[0] = 0

        pltpu.sync_copy(predT.at[0], x1b)
        pltpu.sync_copy(predT.at[1], y1b)
        pltpu.sync_copy(predT.at[2], x2b)
        pltpu.sync_copy(predT.at[3], y2b)
        pltpu.sync_copy(predT.at[4], objb)

        @plsc.parallel_loop(0, P, 16, unroll=4)
        def _box_g(i):
            sl = pl.ds(i, 16)
            cx = x1b[sl]
            cy = y1b[sl]
            w = x2b[sl]
            h = y2b[sl]
            xx1 = jnp.clip(cx - w / 2.0, 0.0, 1.0)
            yy1 = jnp.clip(cy - h / 2.0, 0.0, 1.0)
            xx2 = jnp.clip(cx + w / 2.0, 0.0, 1.0)
            yy2 = jnp.clip(cy + h / 2.0, 0.0, 1.0)
            area = jnp.maximum(xx2 - xx1, 0.0) * jnp.maximum(yy2 - yy1, 0.0)
            x1b[sl] = xx1
            y1b[sl] = yy1
            x2b[sl] = xx2
            y2b[sl] = yy2
            a2b[sl] = area

        args = (predT, kb_hbm, x1b, y1b, x2b, y2b, a2b, objb, sb, kbb, fbuf,
                tuple(cbufs[:6]), tuple(cbufs[6:12]))

        plsc.subcore_barrier()
        ncls = C // 2

        def cond(j):
            return j < ncls

        def body(j):
            _class_nms(core * ncls + j, *args)
            return plsc.fetch_and_add(work.at[0], 1, subcore_id=0)

        lax.while_loop(cond, body,
                       plsc.fetch_and_add(work.at[0], 1, subcore_id=0))

    return sc_nms


_sc_nms = _make_sc_nms()


def kernel(pred, device):
    del device
    predT = jnp.transpose(pred)
    predT = jnp.pad(predT, ((0, 0), (0, P - N)))
    kb8 = _sc_nms(predT)
    labels = jnp.broadcast_to(jnp.arange(C, dtype=jnp.int32)[:, None], (C, K))
    return (kb8[:, :, :4], labels, kb8[:, :, 4], kb8[:, :, 5].astype(bool))
